# baseline (speedup 1.0000x reference)
import jax
import jax.numpy as jnp
from jax import lax
from jax.experimental import pallas as pl
from jax.experimental.pallas import tpu as pltpu
from jax.experimental.pallas import tpu_sc as plsc

BATCH = 4096
SEQ = 50
EMBED = 32
VOCAB = 1000001
VPAD = 1000064
DCH = 1664
DGRID = VPAD // DCH
_info = plsc.get_sparse_core_info()
NC, NS = _info.num_cores, _info.num_subcores
NW = NC * NS
SPW = BATCH // NW
BS = 16
G = SPW // BS


def _detile_body(t_ref, o_ref):
    eye = (
        lax.broadcasted_iota(jnp.int32, (EMBED, EMBED), 0)
        == lax.broadcasted_iota(jnp.int32, (EMBED, EMBED), 1)
    ).astype(jnp.float32)
    o_ref[...] = lax.dot_general(
        t_ref[...],
        eye,
        dimension_numbers=(((0,), (0,)), ((), ())),
        preferred_element_type=jnp.float32,
        precision=lax.Precision.HIGHEST,
    )


def _detile(t_t):
    return pl.pallas_call(
        _detile_body,
        grid=(DGRID,),
        in_specs=[pl.BlockSpec((EMBED, DCH), lambda j: (0, j))],
        out_specs=pl.BlockSpec((DCH, EMBED), lambda j: (j, 0)),
        out_shape=jax.ShapeDtypeStruct((VPAD, EMBED), jnp.float32),
    )(t_t)


def _gather_body(idx_hbm, table_hbm, out_hbm, idx_v, rows_v, gsem):
    w = lax.axis_index("s") * NC + lax.axis_index("c")
    s0 = w * SPW
    pltpu.sync_copy(idx_hbm.at[pl.ds(s0, SPW)], idx_v)

    @pl.loop(0, G)
    def _group(g):
        descs = [
            pltpu.async_copy(
                table_hbm.at[idx_v.at[g * BS + j]],
                rows_v.at[j],
                gsem,
            )
            for j in range(BS)
        ]
        for d in descs:
            d.wait()
        pltpu.sync_copy(rows_v, out_hbm.at[pl.ds(s0 + g * BS, BS)])


def _gather(idx, table_lin):
    mesh = plsc.VectorSubcoreMesh(core_axis_name="c", subcore_axis_name="s")
    f = pl.kernel(
        _gather_body,
        out_type=jax.ShapeDtypeStruct((BATCH, SEQ, EMBED), jnp.float32),
        mesh=mesh,
        scratch_types=[
            pltpu.VMEM((SPW, SEQ), jnp.int32),
            pltpu.VMEM((BS, SEQ, EMBED), jnp.float32),
            pltpu.SemaphoreType.DMA,
        ],
        compiler_params=pltpu.CompilerParams(use_tc_tiling_on_sc=False),
    )
    return f(idx, table_lin)


@jax.jit
def _run(scentences, table):
    table_lin = _detile(table.T)
    return _gather(scentences.astype(jnp.int32), table_lin)


def kernel(scentences, table):
    return _run(scentences, table)


# speedup vs baseline: 1.7180x; 1.7180x over previous
import jax
import jax.numpy as jnp
from jax import lax
from jax.experimental import pallas as pl
from jax.experimental.pallas import tpu as pltpu
from jax.experimental.pallas import tpu_sc as plsc

BATCH = 4096
SEQ = 50
EMBED = 32
_info = plsc.get_sparse_core_info()
NC, NS = _info.num_cores, _info.num_subcores
NW = NC * NS
SPW = BATCH // NW
BS = 16
G = SPW // BS


def _body(idx_hbm, table_hbm, out_hbm, idx_v, rows_v, gsem):
    w = lax.axis_index("s") * NC + lax.axis_index("c")
    s0 = w * SPW
    pltpu.sync_copy(idx_hbm.at[pl.ds(s0, SPW)], idx_v)

    @pl.loop(0, G)
    def _group(g):
        descs = [
            pltpu.async_copy(
                table_hbm.at[idx_v.at[g * BS + j]],
                rows_v.at[j],
                gsem,
            )
            for j in range(BS)
        ]
        for d in descs:
            d.wait()
        pltpu.sync_copy(rows_v, out_hbm.at[pl.ds(s0 + g * BS, BS)])


@jax.jit
def _gather(idx, table):
    mesh = plsc.VectorSubcoreMesh(core_axis_name="c", subcore_axis_name="s")
    f = pl.kernel(
        _body,
        out_type=jax.ShapeDtypeStruct((BATCH, SEQ, EMBED), jnp.float32),
        mesh=mesh,
        scratch_types=[
            pltpu.VMEM((SPW, SEQ), jnp.int32),
            pltpu.VMEM((BS, SEQ, EMBED), jnp.float32),
            pltpu.SemaphoreType.DMA,
        ],
        compiler_params=pltpu.CompilerParams(use_tc_tiling_on_sc=False),
    )
    return f(idx, table)


def kernel(scentences, table):
    return _gather(scentences.astype(jnp.int32), table)
